# baseline (speedup 1.0000x reference)
import jax
import jax.numpy as jnp
from jax import lax
from jax.experimental import pallas as pl
from jax.experimental.pallas import tpu as pltpu, tpu_sc as plsc
from jax._src.pallas import mpmd

_SEQ, _HID = 8192, 1024
_NC, _NS = 2, 16
_ROWS_PER_C = _SEQ // _NC

_SP_CHUNK = 160
_SP_NCHUNK = 8
_SP_NBUF = 3
_SP_ROWS = _SP_CHUNK * _SP_NCHUNK

_ST_ROWS = _ROWS_PER_C - _SP_ROWS
_ST_PER_T = _ST_ROWS // _NS
_ST_CHUNKS = (48, 48, 48, 32)
_ST_BUF = 48

_scalar_mesh = plsc.ScalarSubcoreMesh(axis_name="c", num_cores=_NC)
_vector_mesh = plsc.VectorSubcoreMesh(core_axis_name="c", subcore_axis_name="s")


def _tec_fn(table_hbm, out_hbm, buf0, buf1, isem0, isem1, osem0, osem1,
            *_sp_refs):
    cid = lax.axis_index("c")
    sid = lax.axis_index("s")
    base = cid * _ROWS_PER_C + _SP_ROWS + sid * _ST_PER_T
    bufs = (buf0, buf1)
    isems = (isem0, isem1)
    osems = (osem0, osem1)
    n = len(_ST_CHUNKS)
    in_c = []
    out_c = []
    off = 0
    for j, rows in enumerate(_ST_CHUNKS):
        b = j % 2
        src = table_hbm.at[pl.ds(base + off, rows)]
        dst = out_hbm.at[pl.ds(base + off, rows)]
        buf = bufs[b] if rows == _ST_BUF else bufs[b].at[pl.ds(0, rows)]
        in_c.append(pltpu.make_async_copy(src, buf, isems[b]))
        out_c.append(pltpu.make_async_copy(buf, dst, osems[b]))
        off += rows
    in_c[0].start()
    for j in range(n):
        if j >= 1:
            out_c[j - 1].wait()
        if j + 1 < n:
            in_c[j + 1].start()
        in_c[j].wait()
        out_c[j].start()
    out_c[n - 1].wait()


def _scs_fn(table_hbm, out_hbm, _b0, _b1, _i0, _i1, _o0, _o1, *sp_refs):
    sp_bufs = sp_refs[:_SP_NBUF]
    sp_isems = sp_refs[_SP_NBUF : 2 * _SP_NBUF]
    sp_osems = sp_refs[2 * _SP_NBUF :]
    cid = lax.axis_index("c")
    base = cid * _ROWS_PER_C
    in_c = []
    out_c = []
    for j in range(_SP_NCHUNK):
        b = j % _SP_NBUF
        src = table_hbm.at[pl.ds(base + j * _SP_CHUNK, _SP_CHUNK)]
        dst = out_hbm.at[pl.ds(base + j * _SP_CHUNK, _SP_CHUNK)]
        in_c.append(pltpu.make_async_copy(src, sp_bufs[b], sp_isems[b]))
        out_c.append(pltpu.make_async_copy(sp_bufs[b], dst, sp_osems[b]))
    for j in range(_SP_NBUF):
        in_c[j].start()
    for j in range(_SP_NCHUNK):
        if j >= _SP_NBUF:
            out_c[j - _SP_NBUF].wait()
            in_c[j].start()
        in_c[j].wait()
        out_c[j].start()
    for j in range(_SP_NCHUNK - _SP_NBUF, _SP_NCHUNK):
        out_c[j].wait()


def _make_sc_copy():
    tec_vmem = pltpu.MemorySpace.VMEM @ _vector_mesh
    tec_sem = pltpu.SemaphoreType.DMA @ _vector_mesh
    scs_sem = pltpu.SemaphoreType.DMA @ _scalar_mesh
    scratch = (
        [tec_vmem((_ST_BUF, _HID), jnp.float32) for _ in range(2)]
        + [tec_sem for _ in range(4)]
        + [pltpu.VMEM_SHARED((_SP_CHUNK, _HID), jnp.float32)] * _SP_NBUF
        + [scs_sem for _ in range(2 * _SP_NBUF)]
    )
    return mpmd.mpmd_map(
        [(_scalar_mesh, _scs_fn), (_vector_mesh, _tec_fn)],
        out_types=jax.ShapeDtypeStruct((_SEQ, _HID), jnp.float32),
        scratch_types=scratch,
    )


def kernel(input_ids, table):
    return _make_sc_copy()(table)[None]
